# baseline (speedup 1.0000x reference)
import functools

import jax
import jax.numpy as jnp
from jax import lax
from jax.experimental import pallas as pl
from jax.experimental.pallas import tpu as pltpu
from jax.experimental.pallas import tpu_sc as plsc

N = 10000
E = 160000
NC, NS = 2, 16
NW = NC * NS
C = 128
K = 40
EW = K * C
EP = NW * EW
KG0, KG1 = 64, 16
GCH = NS * (KG0 + KG1)
GPADROWS = GCH + (KG0 - KG1)


def _sc_mesh():
    return plsc.VectorSubcoreMesh(
        core_axis_name="c", subcore_axis_name="s", num_cores=NC, num_subcores=NS)


@functools.cache
def _make_sc_gather():
    @functools.partial(
        pl.kernel,
        mesh=_sc_mesh(),
        out_type=[jax.ShapeDtypeStruct((EP, 128), jnp.float32),
                  jax.ShapeDtypeStruct((EP, 128), jnp.float32)],
        scratch_types=[
            pltpu.VMEM((KG0, C), jnp.int32),
            pltpu.VMEM((KG0, C), jnp.int32),
            pltpu.VMEM((2, C, 128), jnp.float32),
            pltpu.VMEM((2, C, 128), jnp.float32),
            pltpu.SemaphoreType.DMA,
            pltpu.SemaphoreType.DMA,
            pltpu.SemaphoreType.DMA,
            pltpu.SemaphoreType.DMA,
            pltpu.SemaphoreType.DMA,
            pltpu.SemaphoreType.DMA,
            pltpu.SemaphoreType.DMA,
            pltpu.SemaphoreType.DMA,
        ],
        name="sc_gather_packed",
    )
    def k(p_hbm, dsti_hbm, srci_hbm, pd_out, ps_out,
          dv, sv, bufd, bufs, gd0, gs0, gd1, gs1, wd0, ws0, wd1, ws1):
        cid = lax.axis_index("c")
        sid = lax.axis_index("s")
        start = (KG0 + KG1) * sid + cid * KG0
        nch = jnp.where(cid == 0, KG0, KG1)
        base = start * C
        pltpu.sync_copy(dsti_hbm.at[pl.ds(start, KG0)], dv)
        pltpu.sync_copy(srci_hbm.at[pl.ds(start, KG0)], sv)
        gsem = ((gd0, gs0), (gd1, gs1))
        wsem = ((wd0, ws0), (wd1, ws1))

        def fire(j, slot):
            pltpu.async_copy(p_hbm.at[dv.at[j]], bufd.at[slot], gsem[slot][0])
            pltpu.async_copy(p_hbm.at[sv.at[j]], bufs.at[slot], gsem[slot][1])

        def wait_gather(j, slot):
            pltpu.make_async_copy(
                p_hbm.at[dv.at[j]], bufd.at[slot], gsem[slot][0]).wait()
            pltpu.make_async_copy(
                p_hbm.at[sv.at[j]], bufs.at[slot], gsem[slot][1]).wait()

        def write_start(j, slot):
            rows = pl.ds(base + j * C, C)
            pltpu.async_copy(bufd.at[slot], pd_out.at[rows], wsem[slot][0])
            pltpu.async_copy(bufs.at[slot], ps_out.at[rows], wsem[slot][1])

        def write_wait(j, slot):
            rows = pl.ds(base + j * C, C)
            pltpu.make_async_copy(
                bufd.at[slot], pd_out.at[rows], wsem[slot][0]).wait()
            pltpu.make_async_copy(
                bufs.at[slot], ps_out.at[rows], wsem[slot][1]).wait()

        fire(0, 0)
        fire(1, 1)

        def body(t, _):
            g = t * 2
            for slot in (0, 1):
                j = g + slot
                wait_gather(j, slot)
                write_start(j, slot)
            for slot in (0, 1):
                write_wait(g + slot, slot)
                fire(g + 2 + slot, slot)
            return 0

        lax.fori_loop(0, nch // 2 - 1, body, 0)
        for slot in (0, 1):
            j = nch - 2 + slot
            wait_gather(j, slot)
            write_start(j, slot)
        for slot in (0, 1):
            write_wait(nch - 2 + slot, slot)

    return k


def _sc_gather(P, dst_g, src_g):
    return _make_sc_gather()(P, dst_g, src_g)


_BN = 400
_BE = 1600


def _full(shape):
    return pl.BlockSpec(shape, lambda i: tuple(0 for _ in shape))


def _init_body(nf_ref, wi1, bi1, wi2, bi2, h_ref, p_ref):
    nf = nf_ref[...]
    t = jnp.maximum(
        jnp.dot(nf, wi1[...], preferred_element_type=jnp.float32) + bi1[...], 0.0)
    h = jnp.maximum(
        jnp.dot(t, wi2[...], preferred_element_type=jnp.float32) + bi2[...], 0.0)
    h_ref[...] = h
    p_ref[...] = jnp.concatenate(
        [nf, h, jnp.zeros((nf.shape[0], 62), jnp.float32)], axis=1)


def _tc_init(nf, wi1, bi1, wi2, bi2):
    blk = lambda c: pl.BlockSpec((_BN, c), lambda i: (i, 0))
    return pl.pallas_call(
        _init_body,
        grid=(N // _BN,),
        in_specs=[blk(2), _full((2, 128)), _full((1, 128)),
                  _full((128, 64)), _full((1, 64))],
        out_specs=[blk(64), blk(128)],
        out_shape=[jax.ShapeDtypeStruct((N, 64), jnp.float32),
                   jax.ShapeDtypeStruct((N, 128), jnp.float32)],
    )(nf, wi1, bi1, wi2, bi2)


def _edge_body(pd_ref, ps_ref, d_ref, we1, be1, we2, be2, out_ref):
    e_in = jnp.concatenate(
        [pd_ref[:, 0:66], ps_ref[:, 0:66], d_ref[...]], axis=1)
    h1 = jnp.maximum(
        jnp.dot(e_in, we1[...], preferred_element_type=jnp.float32) + be1[...],
        0.0)
    out_ref[...] = jnp.maximum(
        jnp.dot(h1, we2[...], preferred_element_type=jnp.float32) + be2[...],
        0.0)


def _tc_edge(pd, ps, d2, we1, be1, we2, be2):
    return pl.pallas_call(
        _edge_body,
        grid=(E // _BE,),
        in_specs=[pl.BlockSpec((_BE, 128), lambda i: (i, 0)),
                  pl.BlockSpec((_BE, 128), lambda i: (i, 0)),
                  pl.BlockSpec((_BE, 1), lambda i: (i, 0)),
                  _full((133, 128)), _full((1, 128)),
                  _full((128, 64)), _full((1, 64))],
        out_specs=pl.BlockSpec((_BE, 64), lambda i: (i, 0)),
        out_shape=jax.ShapeDtypeStruct((E, 64), jnp.float32),
    )(pd, ps, d2, we1, be1, we2, be2)


def _node_body(h_ref, nf_ref, agg_ref, wn1, bn1, wn2, bn2, h_out, p_out):
    nf = nf_ref[...]
    n_in = jnp.concatenate([h_ref[...], nf, agg_ref[...]], axis=1)
    t = jnp.maximum(
        jnp.dot(n_in, wn1[...], preferred_element_type=jnp.float32) + bn1[...],
        0.0)
    hn = jnp.maximum(
        jnp.dot(t, wn2[...], preferred_element_type=jnp.float32) + bn2[...],
        0.0)
    h_out[...] = hn
    p_out[...] = jnp.concatenate(
        [nf, hn, jnp.zeros((nf.shape[0], 62), jnp.float32)], axis=1)


def _tc_node(h, nf, agg, wn1, bn1, wn2, bn2):
    blk = lambda c: pl.BlockSpec((_BN, c), lambda i: (i, 0))
    return pl.pallas_call(
        _node_body,
        grid=(N // _BN,),
        in_specs=[blk(64), blk(2), blk(64), _full((130, 128)),
                  _full((1, 128)), _full((128, 64)), _full((1, 64))],
        out_specs=[blk(64), blk(128)],
        out_shape=[jax.ShapeDtypeStruct((N, 64), jnp.float32),
                   jax.ShapeDtypeStruct((N, 128), jnp.float32)],
    )(h, nf, agg, wn1, bn1, wn2, bn2)


def _head_body(pd_ref, ps_ref, d_ref, wc1, bc1, wc2, bc2, out_ref):
    c_in = jnp.concatenate(
        [pd_ref[:, 0:66], ps_ref[:, 0:66], d_ref[...]], axis=1)
    c_hid = jnp.maximum(
        jnp.dot(c_in, wc1[...], preferred_element_type=jnp.float32) + bc1[...],
        0.0)
    z = jnp.dot(c_hid, wc2[...], preferred_element_type=jnp.float32) + bc2[...]
    out_ref[...] = jax.nn.sigmoid(z)


def _tc_head(pd, ps, d2, wc1, bc1, wc2, bc2):
    return pl.pallas_call(
        _head_body,
        grid=(E // _BE,),
        in_specs=[pl.BlockSpec((_BE, 128), lambda i: (i, 0)),
                  pl.BlockSpec((_BE, 128), lambda i: (i, 0)),
                  pl.BlockSpec((_BE, 1), lambda i: (i, 0)),
                  _full((133, 64)), _full((1, 64)),
                  _full((64, 1)), _full((1, 1))],
        out_specs=pl.BlockSpec((_BE, 1), lambda i: (i, 0)),
        out_shape=jax.ShapeDtypeStruct((E, 1), jnp.float32),
    )(pd, ps, d2, wc1, bc1, wc2, bc2)


def kernel(node_features, edge_index, distance, Wi1, bi1, Wi2, bi2,
           We1, be1, We2, be2, Wn1, bn1, Wn2, bn2, Wc1, bc1, Wc2, bc2):
    src = edge_index[0].astype(jnp.int32)
    dst = edge_index[1].astype(jnp.int32)
    pad = EP - E
    gpad = ((0, GPADROWS - GCH), (0, 0))
    src_g = jnp.pad(jnp.concatenate(
        [src, jnp.zeros((pad,), jnp.int32)]).reshape(GCH, C), gpad)
    dst_g = jnp.pad(jnp.concatenate(
        [dst, jnp.zeros((pad,), jnp.int32)]).reshape(GCH, C), gpad)
    d2 = distance[:, None]
    bi1r, bi2r = bi1[None, :], bi2[None, :]
    be1r, be2r = be1[None, :], be2[None, :]
    bn1r, bn2r = bn1[None, :], bn2[None, :]
    bc1r, bc2r = bc1[None, :], bc2[None, :]

    h, P = _tc_init(node_features, Wi1, bi1r, Wi2, bi2r)

    perm = jnp.argsort(dst, stable=True)
    dst_sorted = dst[perm]

    def it(_, carry):
        h, P = carry
        pd, ps = _sc_gather(P, dst_g, src_g)
        ehid = _tc_edge(pd, ps, d2, We1, be1r, We2, be2r)
        agg = jax.ops.segment_sum(ehid[perm], dst_sorted, num_segments=N,
                                  indices_are_sorted=True)
        h, P = _tc_node(h, node_features, agg, Wn1, bn1r, Wn2, bn2r)
        return (h, P)

    h, P = lax.fori_loop(0, 16, it, (h, P))

    pd, ps = _sc_gather(P, dst_g, src_g)
    return _tc_head(pd, ps, d2, Wc1, bc1r, Wc2, bc2r)


# speedup vs baseline: 1.0749x; 1.0749x over previous
import functools

import jax
import jax.numpy as jnp
from jax import lax
from jax.experimental import pallas as pl
from jax.experimental.pallas import tpu as pltpu
from jax.experimental.pallas import tpu_sc as plsc

N = 10000
E = 160000
NC, NS = 2, 16
NW = NC * NS
C = 128
K = 40
EW = K * C
EP = NW * EW
KG0, KG1 = 56, 24
GCH = NS * (KG0 + KG1)
GPADROWS = GCH + (KG0 - KG1)


def _sc_mesh():
    return plsc.VectorSubcoreMesh(
        core_axis_name="c", subcore_axis_name="s", num_cores=NC, num_subcores=NS)


@functools.cache
def _make_sc_gather():
    @functools.partial(
        pl.kernel,
        mesh=_sc_mesh(),
        out_type=[jax.ShapeDtypeStruct((EP, 128), jnp.float32),
                  jax.ShapeDtypeStruct((EP, 128), jnp.float32)],
        scratch_types=[
            pltpu.VMEM((KG0, C), jnp.int32),
            pltpu.VMEM((KG0, C), jnp.int32),
            pltpu.VMEM((2, C, 128), jnp.float32),
            pltpu.VMEM((2, C, 128), jnp.float32),
            pltpu.SemaphoreType.DMA,
            pltpu.SemaphoreType.DMA,
            pltpu.SemaphoreType.DMA,
            pltpu.SemaphoreType.DMA,
            pltpu.SemaphoreType.DMA,
            pltpu.SemaphoreType.DMA,
            pltpu.SemaphoreType.DMA,
            pltpu.SemaphoreType.DMA,
        ],
        name="sc_gather_packed",
    )
    def k(p_hbm, dsti_hbm, srci_hbm, pd_out, ps_out,
          dv, sv, bufd, bufs, gd0, gs0, gd1, gs1, wd0, ws0, wd1, ws1):
        cid = lax.axis_index("c")
        sid = lax.axis_index("s")
        start = (KG0 + KG1) * sid + cid * KG0
        nch = jnp.where(cid == 0, KG0, KG1)
        base = start * C
        pltpu.sync_copy(dsti_hbm.at[pl.ds(start, KG0)], dv)
        pltpu.sync_copy(srci_hbm.at[pl.ds(start, KG0)], sv)
        gsem = ((gd0, gs0), (gd1, gs1))
        wsem = ((wd0, ws0), (wd1, ws1))

        def fire(j, slot):
            pltpu.async_copy(p_hbm.at[dv.at[j]], bufd.at[slot], gsem[slot][0])
            pltpu.async_copy(p_hbm.at[sv.at[j]], bufs.at[slot], gsem[slot][1])

        def wait_gather(j, slot):
            pltpu.make_async_copy(
                p_hbm.at[dv.at[j]], bufd.at[slot], gsem[slot][0]).wait()
            pltpu.make_async_copy(
                p_hbm.at[sv.at[j]], bufs.at[slot], gsem[slot][1]).wait()

        def write_start(j, slot):
            rows = pl.ds(base + j * C, C)
            pltpu.async_copy(bufd.at[slot], pd_out.at[rows], wsem[slot][0])
            pltpu.async_copy(bufs.at[slot], ps_out.at[rows], wsem[slot][1])

        def write_wait(j, slot):
            rows = pl.ds(base + j * C, C)
            pltpu.make_async_copy(
                bufd.at[slot], pd_out.at[rows], wsem[slot][0]).wait()
            pltpu.make_async_copy(
                bufs.at[slot], ps_out.at[rows], wsem[slot][1]).wait()

        fire(0, 0)
        fire(1, 1)

        def body(t, _):
            g = t * 2
            for slot in (0, 1):
                j = g + slot
                wait_gather(j, slot)
                write_start(j, slot)
            for slot in (0, 1):
                write_wait(g + slot, slot)
                fire(g + 2 + slot, slot)
            return 0

        lax.fori_loop(0, nch // 2 - 1, body, 0)
        for slot in (0, 1):
            j = nch - 2 + slot
            wait_gather(j, slot)
            write_start(j, slot)
        for slot in (0, 1):
            write_wait(nch - 2 + slot, slot)

    return k


def _sc_gather(P, dst_g, src_g):
    return _make_sc_gather()(P, dst_g, src_g)


_BN = 400
_BE = 1600


def _full(shape):
    return pl.BlockSpec(shape, lambda i: tuple(0 for _ in shape))


def _init_body(nf_ref, wi1, bi1, wi2, bi2, h_ref, p_ref):
    nf = nf_ref[...]
    t = jnp.maximum(
        jnp.dot(nf, wi1[...], preferred_element_type=jnp.float32) + bi1[...], 0.0)
    h = jnp.maximum(
        jnp.dot(t, wi2[...], preferred_element_type=jnp.float32) + bi2[...], 0.0)
    h_ref[...] = h
    p_ref[...] = jnp.concatenate(
        [nf, h, jnp.zeros((nf.shape[0], 62), jnp.float32)], axis=1)


def _tc_init(nf, wi1, bi1, wi2, bi2):
    blk = lambda c: pl.BlockSpec((_BN, c), lambda i: (i, 0))
    return pl.pallas_call(
        _init_body,
        grid=(N // _BN,),
        in_specs=[blk(2), _full((2, 128)), _full((1, 128)),
                  _full((128, 64)), _full((1, 64))],
        out_specs=[blk(64), blk(128)],
        out_shape=[jax.ShapeDtypeStruct((N, 64), jnp.float32),
                   jax.ShapeDtypeStruct((N, 128), jnp.float32)],
    )(nf, wi1, bi1, wi2, bi2)


def _edge_body(pd_ref, ps_ref, d_ref, we1, be1, we2, be2, out_ref):
    e_in = jnp.concatenate(
        [pd_ref[:, 0:66], ps_ref[:, 0:66], d_ref[...]], axis=1)
    h1 = jnp.maximum(
        jnp.dot(e_in, we1[...], preferred_element_type=jnp.float32) + be1[...],
        0.0)
    out_ref[...] = jnp.maximum(
        jnp.dot(h1, we2[...], preferred_element_type=jnp.float32) + be2[...],
        0.0)


def _tc_edge(pd, ps, d2, we1, be1, we2, be2):
    return pl.pallas_call(
        _edge_body,
        grid=(E // _BE,),
        in_specs=[pl.BlockSpec((_BE, 128), lambda i: (i, 0)),
                  pl.BlockSpec((_BE, 128), lambda i: (i, 0)),
                  pl.BlockSpec((_BE, 1), lambda i: (i, 0)),
                  _full((133, 128)), _full((1, 128)),
                  _full((128, 64)), _full((1, 64))],
        out_specs=pl.BlockSpec((_BE, 64), lambda i: (i, 0)),
        out_shape=jax.ShapeDtypeStruct((E, 64), jnp.float32),
    )(pd, ps, d2, we1, be1, we2, be2)


def _node_body(h_ref, nf_ref, agg_ref, wn1, bn1, wn2, bn2, h_out, p_out):
    nf = nf_ref[...]
    n_in = jnp.concatenate([h_ref[...], nf, agg_ref[...]], axis=1)
    t = jnp.maximum(
        jnp.dot(n_in, wn1[...], preferred_element_type=jnp.float32) + bn1[...],
        0.0)
    hn = jnp.maximum(
        jnp.dot(t, wn2[...], preferred_element_type=jnp.float32) + bn2[...],
        0.0)
    h_out[...] = hn
    p_out[...] = jnp.concatenate(
        [nf, hn, jnp.zeros((nf.shape[0], 62), jnp.float32)], axis=1)


def _tc_node(h, nf, agg, wn1, bn1, wn2, bn2):
    blk = lambda c: pl.BlockSpec((_BN, c), lambda i: (i, 0))
    return pl.pallas_call(
        _node_body,
        grid=(N // _BN,),
        in_specs=[blk(64), blk(2), blk(64), _full((130, 128)),
                  _full((1, 128)), _full((128, 64)), _full((1, 64))],
        out_specs=[blk(64), blk(128)],
        out_shape=[jax.ShapeDtypeStruct((N, 64), jnp.float32),
                   jax.ShapeDtypeStruct((N, 128), jnp.float32)],
    )(h, nf, agg, wn1, bn1, wn2, bn2)


def _head_body(pd_ref, ps_ref, d_ref, wc1, bc1, wc2, bc2, out_ref):
    c_in = jnp.concatenate(
        [pd_ref[:, 0:66], ps_ref[:, 0:66], d_ref[...]], axis=1)
    c_hid = jnp.maximum(
        jnp.dot(c_in, wc1[...], preferred_element_type=jnp.float32) + bc1[...],
        0.0)
    z = jnp.dot(c_hid, wc2[...], preferred_element_type=jnp.float32) + bc2[...]
    out_ref[...] = jax.nn.sigmoid(z)


def _tc_head(pd, ps, d2, wc1, bc1, wc2, bc2):
    return pl.pallas_call(
        _head_body,
        grid=(E // _BE,),
        in_specs=[pl.BlockSpec((_BE, 128), lambda i: (i, 0)),
                  pl.BlockSpec((_BE, 128), lambda i: (i, 0)),
                  pl.BlockSpec((_BE, 1), lambda i: (i, 0)),
                  _full((133, 64)), _full((1, 64)),
                  _full((64, 1)), _full((1, 1))],
        out_specs=pl.BlockSpec((_BE, 1), lambda i: (i, 0)),
        out_shape=jax.ShapeDtypeStruct((E, 1), jnp.float32),
    )(pd, ps, d2, wc1, bc1, wc2, bc2)


def kernel(node_features, edge_index, distance, Wi1, bi1, Wi2, bi2,
           We1, be1, We2, be2, Wn1, bn1, Wn2, bn2, Wc1, bc1, Wc2, bc2):
    src = edge_index[0].astype(jnp.int32)
    dst = edge_index[1].astype(jnp.int32)
    pad = EP - E
    gpad = ((0, GPADROWS - GCH), (0, 0))
    src_g = jnp.pad(jnp.concatenate(
        [src, jnp.zeros((pad,), jnp.int32)]).reshape(GCH, C), gpad)
    dst_g = jnp.pad(jnp.concatenate(
        [dst, jnp.zeros((pad,), jnp.int32)]).reshape(GCH, C), gpad)
    d2 = distance[:, None]
    bi1r, bi2r = bi1[None, :], bi2[None, :]
    be1r, be2r = be1[None, :], be2[None, :]
    bn1r, bn2r = bn1[None, :], bn2[None, :]
    bc1r, bc2r = bc1[None, :], bc2[None, :]

    h, P = _tc_init(node_features, Wi1, bi1r, Wi2, bi2r)

    def it(_, carry):
        h, P = carry
        pd, ps = _sc_gather(P, dst_g, src_g)
        ehid = _tc_edge(pd, ps, d2, We1, be1r, We2, be2r)
        agg = jax.ops.segment_sum(ehid, dst, num_segments=N)
        h, P = _tc_node(h, node_features, agg, Wn1, bn1r, Wn2, bn2r)
        return (h, P)

    h, P = lax.fori_loop(0, 16, it, (h, P))

    pd, ps = _sc_gather(P, dst_g, src_g)
    return _tc_head(pd, ps, d2, Wc1, bc1r, Wc2, bc2r)
